# baseline (speedup 1.0000x reference)
import jax
import jax.numpy as jnp
from jax.experimental import pallas as pl


TILE_N = 512


def _chamfer_body(a_ref, b_ref, sq2_ref, dist1_ref, dist2_ref):
    i = pl.program_id(1)
    a = a_ref[0]
    bt = b_ref[0]
    sq1_col = a[:, 3:4]
    sq2_row = sq2_ref[0]
    dot = jax.lax.dot_general(
        a.astype(jnp.bfloat16), bt.astype(jnp.bfloat16),
        (((1,), (1,)), ((), ())),
        preferred_element_type=jnp.float32,
    )
    d = (sq1_col + sq2_row) - 2.0 * dot
    dist1_ref[0, :, :] = jnp.min(d, axis=1, keepdims=True)
    partial = jnp.min(d, axis=0, keepdims=True)

    @pl.when(i == 0)
    def _init():
        dist2_ref[0, :, :] = partial

    @pl.when(i > 0)
    def _acc():
        dist2_ref[0, :, :] = jnp.minimum(dist2_ref[0, :, :], partial)


@jax.jit
def kernel(input1, input2):
    b, n, _ = input1.shape
    m = input2.shape[1]
    sq1 = jnp.sum(input1 * input1, axis=-1)
    sq2 = jnp.sum(input2 * input2, axis=-1)
    zeros1 = jnp.zeros((b, n, 4), jnp.float32)
    zeros2 = jnp.zeros((b, m, 5), jnp.float32)
    a = jnp.concatenate([input1, sq1[..., None], zeros1], axis=-1)
    bb = jnp.concatenate([input2, zeros2], axis=-1)
    sq2r = sq2[:, None, :]

    grid = (b, n // TILE_N)
    dist1, dist2 = pl.pallas_call(
        _chamfer_body,
        grid=grid,
        in_specs=[
            pl.BlockSpec((1, TILE_N, 8), lambda bi, i: (bi, i, 0)),
            pl.BlockSpec((1, m, 8), lambda bi, i: (bi, 0, 0)),
            pl.BlockSpec((1, 1, m), lambda bi, i: (bi, 0, 0)),
        ],
        out_specs=[
            pl.BlockSpec((1, TILE_N, 1), lambda bi, i: (bi, i, 0)),
            pl.BlockSpec((1, 1, m), lambda bi, i: (bi, 0, 0)),
        ],
        out_shape=[
            jax.ShapeDtypeStruct((b, n, 1), jnp.float32),
            jax.ShapeDtypeStruct((b, 1, m), jnp.float32),
        ],
    )(a, bb, sq2r)
    return dist1[:, :, 0], dist2[:, 0, :]
